# baseline (speedup 1.0000x reference)
import functools

import jax
import jax.numpy as jnp
from jax import lax
from jax.experimental import pallas as pl
from jax.experimental.pallas import tpu as pltpu
from jax.experimental.pallas import tpu_sc as plsc

DIM_MODEL = 768
SEQ_LEN = 2048
BATCH = 4
N_ROWS = BATCH * SEQ_LEN

NUM_WORKERS = 32
POS_PER_W = SEQ_LEN // NUM_WORKERS
CHUNK = POS_PER_W
NCHUNKS = BATCH
LANES = 16
VECS_PER_ROW = DIM_MODEL // LANES

_MESH = plsc.VectorSubcoreMesh(core_axis_name="c", subcore_axis_name="s")

_TWO_PI_HI = 6.28125
_TWO_PI_LO = 1.9353072e-3
_INV_2PI = 0.15915494309189535

_SIN_C = (-1.6666667e-1, 8.3333333e-3, -1.9841270e-4, 2.7557319e-6, -2.5052108e-8)
_COS_C = (-5.0e-1, 4.1666667e-2, -1.3888889e-3, 2.4801587e-5, -2.7557319e-7, 2.0876757e-9)


def _sin_poly(r, r2):
    acc = jnp.float32(_SIN_C[-1])
    for c in _SIN_C[-2::-1]:
        acc = acc * r2 + jnp.float32(c)
    return r * (acc * r2 + jnp.float32(1.0))


def _cos_poly(r2):
    acc = jnp.float32(_COS_C[-1])
    for c in _COS_C[-2::-1]:
        acc = acc * r2 + jnp.float32(c)
    return acc * r2 + jnp.float32(1.0)


@functools.partial(
    pl.kernel,
    mesh=_MESH,
    out_type=jax.ShapeDtypeStruct((N_ROWS, DIM_MODEL), jnp.float32),
    scratch_types=[
        pltpu.VMEM((BATCH * POS_PER_W,), jnp.int32),
        pltpu.VMEM((POS_PER_W, LANES), jnp.float32),
        pltpu.VMEM((CHUNK, DIM_MODEL), jnp.float32),
        pltpu.VMEM((CHUNK, DIM_MODEL), jnp.float32),
        pltpu.SemaphoreType.DMA,
        pltpu.SemaphoreType.DMA,
        pltpu.SemaphoreType.DMA,
        pltpu.SemaphoreType.DMA,
        pltpu.SemaphoreType.DMA,
    ],
)
def _embed_sc(
    x_hbm, table_hbm, out_hbm,
    idx_v, pos_v, rows0, rows1,
    isem, g0, g1, w0, w1,
):
    rows = (rows0, rows1)
    gsem = (g0, g1)
    wsem = (w0, w1)

    wid = lax.axis_index("s") * 2 + lax.axis_index("c")
    pbase = wid * POS_PER_W
    pbase_f = lax.convert_element_type(pbase, jnp.float32)

    idesc = [
        pltpu.async_copy(
            x_hbm.at[b, pl.ds(pbase, POS_PER_W)],
            idx_v.at[pl.ds(b * POS_PER_W, POS_PER_W)],
            isem,
        )
        for b in range(BATCH)
    ]

    def start_gather(ci):
        return pltpu.async_copy(
            table_hbm.at[idx_v.at[pl.ds(ci * POS_PER_W, CHUNK)]],
            rows[ci & 1],
            gsem[ci & 1],
        )

    lane = lax.convert_element_type(lax.iota(jnp.int32, LANES), jnp.float32)
    zero = jnp.float32(0.0)

    def onehot(k):
        return jnp.maximum(jnp.float32(1.0) - jnp.abs(lane - jnp.float32(k)), zero)

    oh0 = onehot(0)
    oh1 = onehot(1)
    parity = lax.convert_element_type(
        lax.rem(lax.iota(jnp.int32, LANES), 2), jnp.float32
    )
    tail = parity - oh1
    scale = (
        jnp.float32(1e-8) * onehot(2)
        + jnp.float32(1e-16) * onehot(4)
        + jnp.float32(1e-24) * onehot(6)
        + jnp.float32(1e-32) * onehot(8)
    )

    def build_row(r):
        p = pbase_f + jnp.float32(r)
        magic = jnp.float32(8388608.0)
        n = (p * jnp.float32(_INV_2PI) + magic) - magic
        red = (p - n * jnp.float32(_TWO_PI_HI)) - n * jnp.float32(_TWO_PI_LO)
        r2 = red * red
        vec = tail + _sin_poly(red, r2) * oh0 + _cos_poly(r2) * oh1 + p * scale
        pos_v[r, pl.ds(0, LANES)] = vec

    for r in range(POS_PER_W):
        build_row(r)

    for d in idesc:
        d.wait()
    gd = {0: start_gather(0)}
    wd = {}
    for ci in range(NCHUNKS):
        buf = ci & 1
        if ci + 1 < NCHUNKS:
            if ci >= 1:
                wd[ci - 1].wait()
            gd[ci + 1] = start_gather(ci + 1)
        gd[ci].wait()
        rv = rows[buf]

        def add_row(r, _):
            plsc.addupdate(rv.at[r, pl.ds(0, LANES)], pos_v[r, pl.ds(0, LANES)])
            for j in range(1, VECS_PER_ROW):
                plsc.addupdate(rv.at[r, pl.ds(j * LANES, LANES)], parity)
            return _

        lax.fori_loop(0, CHUNK, add_row, None)
        wd[ci] = pltpu.async_copy(
            rv,
            out_hbm.at[pl.ds(ci * SEQ_LEN + pbase, CHUNK)],
            wsem[buf],
        )
    wd[NCHUNKS - 2].wait()
    wd[NCHUNKS - 1].wait()


def kernel(x, table):
    xi = x.astype(jnp.int32)
    out = _embed_sc(xi, table)
    return out.reshape(BATCH, SEQ_LEN, DIM_MODEL)


# speedup vs baseline: 1.0077x; 1.0077x over previous
import functools

import jax
import jax.numpy as jnp
from jax import lax
from jax.experimental import pallas as pl
from jax.experimental.pallas import tpu as pltpu
from jax.experimental.pallas import tpu_sc as plsc

DIM_MODEL = 768
SEQ_LEN = 2048
BATCH = 4
N_ROWS = BATCH * SEQ_LEN

NUM_WORKERS = 32
POS_PER_W = SEQ_LEN // NUM_WORKERS
CHUNK = POS_PER_W
NCHUNKS = BATCH
LANES = 16
VECS_PER_ROW = DIM_MODEL // LANES

_MESH = plsc.VectorSubcoreMesh(core_axis_name="c", subcore_axis_name="s")

_TWO_PI_HI = 6.28125
_TWO_PI_LO = 1.9353072e-3
_INV_2PI = 0.15915494309189535

_SIN_C = (-1.6666667e-1, 8.3333333e-3, -1.9841270e-4, 2.7557319e-6, -2.5052108e-8)
_COS_C = (-5.0e-1, 4.1666667e-2, -1.3888889e-3, 2.4801587e-5, -2.7557319e-7, 2.0876757e-9)


def _sin_poly(r, r2):
    acc = jnp.float32(_SIN_C[-1])
    for c in _SIN_C[-2::-1]:
        acc = acc * r2 + jnp.float32(c)
    return r * (acc * r2 + jnp.float32(1.0))


def _cos_poly(r2):
    acc = jnp.float32(_COS_C[-1])
    for c in _COS_C[-2::-1]:
        acc = acc * r2 + jnp.float32(c)
    return acc * r2 + jnp.float32(1.0)


@functools.partial(
    pl.kernel,
    mesh=_MESH,
    out_type=jax.ShapeDtypeStruct((N_ROWS, DIM_MODEL), jnp.float32),
    scratch_types=[
        pltpu.VMEM((BATCH * POS_PER_W,), jnp.int32),
        pltpu.VMEM((POS_PER_W, LANES), jnp.float32),
        pltpu.VMEM((CHUNK, DIM_MODEL), jnp.float32),
        pltpu.VMEM((CHUNK, DIM_MODEL), jnp.float32),
        pltpu.SemaphoreType.DMA,
        pltpu.SemaphoreType.DMA,
        pltpu.SemaphoreType.DMA,
        pltpu.SemaphoreType.DMA,
        pltpu.SemaphoreType.DMA,
    ],
)
def _embed_sc(
    x_hbm, table_hbm, out_hbm,
    idx_v, pos_v, rows0, rows1,
    isem, g0, g1, w0, w1,
):
    rows = (rows0, rows1)
    gsem = (g0, g1)
    wsem = (w0, w1)

    wid = lax.axis_index("s") * 2 + lax.axis_index("c")
    pbase = wid * POS_PER_W
    pbase_f = lax.convert_element_type(pbase, jnp.float32)

    idesc = [
        pltpu.async_copy(
            x_hbm.at[b, pl.ds(pbase, POS_PER_W)],
            idx_v.at[pl.ds(b * POS_PER_W, POS_PER_W)],
            isem,
        )
        for b in range(BATCH)
    ]

    def start_gather(ci):
        return pltpu.async_copy(
            table_hbm.at[idx_v.at[pl.ds(ci * POS_PER_W, CHUNK)]],
            rows[ci & 1],
            gsem[ci & 1],
        )

    lane = lax.convert_element_type(lax.iota(jnp.int32, LANES), jnp.float32)
    zero = jnp.float32(0.0)

    def onehot(k):
        return jnp.maximum(jnp.float32(1.0) - jnp.abs(lane - jnp.float32(k)), zero)

    oh0 = onehot(0)
    oh1 = onehot(1)
    parity = lax.convert_element_type(
        lax.rem(lax.iota(jnp.int32, LANES), 2), jnp.float32
    )
    tail = parity - oh1
    scale = (
        jnp.float32(1e-8) * onehot(2)
        + jnp.float32(1e-16) * onehot(4)
        + jnp.float32(1e-24) * onehot(6)
        + jnp.float32(1e-32) * onehot(8)
    )

    def build_row(r):
        p = pbase_f + jnp.float32(r)
        magic = jnp.float32(8388608.0)
        n = (p * jnp.float32(_INV_2PI) + magic) - magic
        red = (p - n * jnp.float32(_TWO_PI_HI)) - n * jnp.float32(_TWO_PI_LO)
        r2 = red * red
        vec = tail + _sin_poly(red, r2) * oh0 + _cos_poly(r2) * oh1 + p * scale
        pos_v[r, pl.ds(0, LANES)] = vec

    for d in idesc:
        d.wait()
    gd = {0: start_gather(0)}

    for r in range(POS_PER_W):
        build_row(r)

    wd = {}
    for ci in range(NCHUNKS):
        buf = ci & 1
        if ci + 1 < NCHUNKS:
            if ci >= 1:
                wd[ci - 1].wait()
            gd[ci + 1] = start_gather(ci + 1)
        gd[ci].wait()
        rv = rows[buf]

        def add_row(r, _):
            plsc.addupdate(rv.at[r, pl.ds(0, LANES)], pos_v[r, pl.ds(0, LANES)])
            for j in range(1, VECS_PER_ROW):
                plsc.addupdate(rv.at[r, pl.ds(j * LANES, LANES)], parity)
            return _

        lax.fori_loop(0, CHUNK, add_row, None)
        wd[ci] = pltpu.async_copy(
            rv,
            out_hbm.at[pl.ds(ci * SEQ_LEN + pbase, CHUNK)],
            wsem[buf],
        )
    wd[NCHUNKS - 2].wait()
    wd[NCHUNKS - 1].wait()


def kernel(x, table):
    xi = x.astype(jnp.int32)
    out = _embed_sc(xi, table)
    return out.reshape(BATCH, SEQ_LEN, DIM_MODEL)


# speedup vs baseline: 1.1058x; 1.0973x over previous
import functools

import numpy as np
import jax
import jax.numpy as jnp
from jax import lax
from jax.experimental import pallas as pl
from jax.experimental.pallas import tpu as pltpu
from jax.experimental.pallas import tpu_sc as plsc

DIM_MODEL = 768
SEQ_LEN = 2048
BATCH = 4
N_ROWS = BATCH * SEQ_LEN

NUM_WORKERS = 32
POS_PER_W = SEQ_LEN // NUM_WORKERS
CHUNK = POS_PER_W
NCHUNKS = BATCH
LANES = 16
VECS_PER_ROW = DIM_MODEL // LANES


def _pos_head_np() -> np.ndarray:
    position = np.arange(0, SEQ_LEN, dtype=np.float32)[:, None]
    s2i = np.arange(0, DIM_MODEL, 2, dtype=np.float32)
    with np.errstate(over="ignore"):
        denom = np.power(np.float32(10000.0), s2i, dtype=np.float32)
    ratio = (position / denom).astype(np.float32)
    enc = np.zeros((SEQ_LEN, DIM_MODEL), dtype=np.float32)
    enc[:, 0::2] = np.sin(ratio)
    enc[:, 1::2] = np.cos(ratio)
    assert np.all(enc[:, LANES:] == np.tile(np.float32([0.0, 1.0]), DIM_MODEL // 2)[LANES:])
    return np.ascontiguousarray(enc[:, :LANES])


_POS_HEAD = _pos_head_np()

_MESH = plsc.VectorSubcoreMesh(core_axis_name="c", subcore_axis_name="s")


@functools.partial(
    pl.kernel,
    mesh=_MESH,
    out_type=jax.ShapeDtypeStruct((N_ROWS, DIM_MODEL), jnp.float32),
    scratch_types=[
        pltpu.VMEM((BATCH * POS_PER_W,), jnp.int32),
        pltpu.VMEM((POS_PER_W, LANES), jnp.float32),
        pltpu.VMEM((CHUNK, DIM_MODEL), jnp.float32),
        pltpu.VMEM((CHUNK, DIM_MODEL), jnp.float32),
        pltpu.SemaphoreType.DMA,
        pltpu.SemaphoreType.DMA,
        pltpu.SemaphoreType.DMA,
        pltpu.SemaphoreType.DMA,
        pltpu.SemaphoreType.DMA,
        pltpu.SemaphoreType.DMA,
    ],
)
def _embed_sc(
    x_hbm, table_hbm, pos_hbm, out_hbm,
    idx_v, pos_v, rows0, rows1,
    isem, psem, g0, g1, w0, w1,
):
    rows = (rows0, rows1)
    gsem = (g0, g1)
    wsem = (w0, w1)

    wid = lax.axis_index("s") * 2 + lax.axis_index("c")
    pbase = wid * POS_PER_W

    tail = lax.convert_element_type(
        lax.rem(lax.iota(jnp.int32, LANES), 2), jnp.float32
    )

    idesc = [
        pltpu.async_copy(
            x_hbm.at[b, pl.ds(pbase, POS_PER_W)],
            idx_v.at[pl.ds(b * POS_PER_W, POS_PER_W)],
            isem,
        )
        for b in range(BATCH)
    ]
    pdesc = pltpu.async_copy(pos_hbm.at[pl.ds(pbase, POS_PER_W)], pos_v, psem)
    for d in idesc:
        d.wait()

    def start_gather(ci):
        return pltpu.async_copy(
            table_hbm.at[idx_v.at[pl.ds(ci * POS_PER_W, CHUNK)]],
            rows[ci & 1],
            gsem[ci & 1],
        )

    gd = {0: start_gather(0)}
    wd = {}
    for ci in range(NCHUNKS):
        buf = ci & 1
        if ci + 1 < NCHUNKS:
            if ci >= 1:
                wd[ci - 1].wait()
            gd[ci + 1] = start_gather(ci + 1)
        gd[ci].wait()
        if ci == 0:
            pdesc.wait()
        rv = rows[buf]

        def add_row(r, _):
            plsc.addupdate(rv.at[r, pl.ds(0, LANES)], pos_v[r, pl.ds(0, LANES)])
            for j in range(1, VECS_PER_ROW):
                plsc.addupdate(rv.at[r, pl.ds(j * LANES, LANES)], tail)
            return _

        lax.fori_loop(0, CHUNK, add_row, None)
        wd[ci] = pltpu.async_copy(
            rv,
            out_hbm.at[pl.ds(ci * SEQ_LEN + pbase, CHUNK)],
            wsem[buf],
        )
    wd[NCHUNKS - 2].wait()
    wd[NCHUNKS - 1].wait()


def kernel(x, table):
    xi = x.astype(jnp.int32)
    out = _embed_sc(xi, table, _POS_HEAD)
    return out.reshape(BATCH, SEQ_LEN, DIM_MODEL)
